# baseline (speedup 1.0000x reference)
import functools

import jax
import jax.numpy as jnp
from jax import lax
from jax.experimental import pallas as pl
from jax.experimental.pallas import tpu as pltpu
from jax.experimental.pallas import tpu_sc as plsc

N = 65536
NS_TOT = 32768
CARD = 100000
FEAT = 1000
D = 256
CROWS = 4096
REP = 4
NC = 2
NSUB = 16
NW = NC * NSUB
ROWS_PER_W = N // NW
SUB = 64
DEPTH = 3

NWP = 8
JS_PER_P = NS_TOT // NWP
BINS = N // NC // NSUB


def _proj_body(a_ref, w_ref, b_ref, o_ref):
    o_ref[...] = (
        jnp.dot(a_ref[...].astype(jnp.bfloat16),
                w_ref[...].astype(jnp.bfloat16),
                preferred_element_type=jnp.float32)
        + b_ref[...]
    )


def _project(table, w, b2d, m_blk):
    m = table.shape[0]
    return pl.pallas_call(
        _proj_body,
        grid=(m // m_blk,),
        in_specs=[
            pl.BlockSpec((m_blk, D), lambda i: (i, 0)),
            pl.BlockSpec((D, D), lambda i: (0, 0)),
            pl.BlockSpec((1, D), lambda i: (0, 0)),
        ],
        out_specs=pl.BlockSpec((m_blk, D), lambda i: (i, 0)),
        out_shape=jax.ShapeDtypeStruct((m, D), jnp.float32),
    )(table, w, b2d)


def _sc_winner_body(
    idx_line_hbm, val_line_hbm, idx_arc_hbm, val_arc_hbm,
    ptr_hbm,
    winner_v, stage_i_v, stage_v_v, parts_v, ptr_v, spmem_sh, sem_p,
):
    c = lax.axis_index("c")
    s = lax.axis_index("s")
    iota = lax.iota(jnp.int32, 16)
    neg1 = jnp.full((16,), -1, jnp.int32)

    @pl.when(s < NWP)
    def _():
        def init_body(i, _):
            for u in range(8):
                winner_v[pl.ds(i * 128 + u * 16, 16)] = neg1
            return 0

        lax.fori_loop(0, N // 128, init_body, 0)

        jbase = s * JS_PER_P

        def run_table(idx_hbm, val_hbm, table_flag):
            pltpu.sync_copy(idx_hbm.at[pl.ds(jbase, JS_PER_P)], stage_i_v)
            pltpu.sync_copy(val_hbm.at[pl.ds(jbase, JS_PER_P)], stage_v_v)
            tconst = table_flag << 25

            def chunk_body(i, _):
                idx16 = stage_i_v[pl.ds(i * 16, 16)]
                val16 = stage_v_v[pl.ds(i * 16, 16)]
                jvec = jbase + i * 16 + iota
                key = jvec * 1024 + val16 + tconst
                for k in range(16):
                    plsc.store_scatter(winner_v, [idx16], key, mask=iota == k)
                return 0

            lax.fori_loop(0, JS_PER_P // 16, chunk_body, 0)

        run_table(idx_line_hbm, val_line_hbm, 0)
        run_table(idx_arc_hbm, val_arc_hbm, 1)

        pltpu.sync_copy(winner_v, spmem_sh.at[s])

    plsc.subcore_barrier()

    binbase = c * (N // NC) + s * BINS
    for t in range(NWP):
        pltpu.async_copy(
            spmem_sh.at[t, pl.ds(binbase, BINS)], parts_v.at[t], sem_p)
    for t in range(NWP):
        pltpu.make_async_copy(
            spmem_sh.at[t, pl.ds(binbase, BINS)], parts_v.at[t], sem_p).wait()

    def dec_body(i, _):
        sl = pl.ds(i * 16, 16)
        k16 = parts_v[0, sl]
        for t in range(1, NWP):
            k16 = jnp.maximum(k16, parts_v[t, sl])
        tab = lax.shift_right_logical(k16, 25)
        ptr = tab * FEAT + (k16 & 1023)
        zptr = 2000 + ((i * 16) & 2047) + iota
        ptr_v[sl] = jnp.where(k16 < 0, zptr, ptr) + (i & (REP - 1)) * CROWS
        return 0

    lax.fori_loop(0, BINS // 16, dec_body, 0)

    pltpu.sync_copy(ptr_v, ptr_hbm.at[pl.ds(binbase, BINS)])


_sc_winner = functools.partial(
    pl.kernel,
    out_type=jax.ShapeDtypeStruct((N,), jnp.int32),
    mesh=plsc.VectorSubcoreMesh(core_axis_name="c", subcore_axis_name="s"),
    compiler_params=pltpu.CompilerParams(needs_layout_passes=False),
    cost_estimate=pl.CostEstimate(
        flops=4 * NS_TOT, transcendentals=0,
        bytes_accessed=4 * (4 * NS_TOT + 2 * N)),
    scratch_types=[
        pltpu.VMEM((N,), jnp.int32),
        pltpu.VMEM((JS_PER_P,), jnp.int32),
        pltpu.VMEM((JS_PER_P,), jnp.int32),
        pltpu.VMEM((NWP, BINS), jnp.int32),
        pltpu.VMEM((BINS,), jnp.int32),
        pltpu.VMEM_SHARED((NWP, N), jnp.int32),
        pltpu.SemaphoreType.DMA,
    ],
)(_sc_winner_body)


def _sc_emit_body(
    ff_hbm, ptr_hbm, fproj_hbm, cproj_hbm,
    out_hbm,
    ff_v, ptr_v, buf_a, buf_b,
    sem_a0, sem_a1, sem_a2, sem_b0, sem_b1, sem_b2,
    sem_o0, sem_o1, sem_o2,
):
    c = lax.axis_index("c")
    s = lax.axis_index("s")
    w = s * NC + c
    rowbase = w * ROWS_PER_W

    pltpu.sync_copy(ff_hbm.at[pl.ds(rowbase, ROWS_PER_W)], ff_v)
    pltpu.sync_copy(ptr_hbm.at[pl.ds(rowbase, ROWS_PER_W)], ptr_v)

    sems_a = (sem_a0, sem_a1, sem_a2)
    sems_b = (sem_b0, sem_b1, sem_b2)
    sems_o = (sem_o0, sem_o1, sem_o2)
    NSB = ROWS_PER_W // SUB

    def gather_pair(sb, slot):
        pltpu.async_copy(
            fproj_hbm.at[ff_v.at[pl.ds(sb * SUB, SUB)]],
            buf_a.at[slot], sems_a[slot])
        pltpu.async_copy(
            cproj_hbm.at[ptr_v.at[pl.ds(sb * SUB, SUB)]],
            buf_b.at[slot], sems_b[slot])

    def wait_pair(sb, slot):
        pltpu.make_async_copy(
            fproj_hbm.at[ff_v.at[pl.ds(sb * SUB, SUB)]],
            buf_a.at[slot], sems_a[slot]).wait()
        pltpu.make_async_copy(
            cproj_hbm.at[ptr_v.at[pl.ds(sb * SUB, SUB)]],
            buf_b.at[slot], sems_b[slot]).wait()

    def out_desc(sb, slot):
        return pltpu.make_async_copy(
            buf_a.at[slot], out_hbm.at[pl.ds(rowbase + sb * SUB, SUB)],
            sems_o[slot])

    for i in range(DEPTH - 1):
        gather_pair(i, i)
    for sb in range(NSB):
        slot = sb % DEPTH
        wait_pair(sb, slot)
        nxt = sb + DEPTH - 1
        if nxt < NSB:
            tslot = nxt % DEPTH
            if sb >= 1:
                out_desc(sb - 1, tslot).wait()
            gather_pair(nxt, tslot)

        def add_body(r, _, slot=slot):
            for kk in range(D // 16):
                sl = pl.ds(kk * 16, 16)
                plsc.addupdate(buf_a.at[slot, r, sl], buf_b[slot, r, sl])
            return 0

        lax.fori_loop(0, SUB, add_body, 0)
        out_desc(sb, slot).start()
    for m in range(NSB - DEPTH, NSB):
        out_desc(m, m % DEPTH).wait()


_sc_emit = functools.partial(
    pl.kernel,
    out_type=jax.ShapeDtypeStruct((N, D), jnp.float32),
    mesh=plsc.VectorSubcoreMesh(core_axis_name="c", subcore_axis_name="s"),
    compiler_params=pltpu.CompilerParams(needs_layout_passes=False),
    scratch_types=[
        pltpu.VMEM((ROWS_PER_W,), jnp.int32),
        pltpu.VMEM((ROWS_PER_W,), jnp.int32),
        pltpu.VMEM((DEPTH, SUB, D), jnp.float32),
        pltpu.VMEM((DEPTH, SUB, D), jnp.float32),
        pltpu.SemaphoreType.DMA,
        pltpu.SemaphoreType.DMA,
        pltpu.SemaphoreType.DMA,
        pltpu.SemaphoreType.DMA,
        pltpu.SemaphoreType.DMA,
        pltpu.SemaphoreType.DMA,
        pltpu.SemaphoreType.DMA,
        pltpu.SemaphoreType.DMA,
        pltpu.SemaphoreType.DMA,
    ],
)(_sc_emit_body)


def kernel(fixed_features, sparse_index_line, sparse_value_line,
           sparse_index_arc, sparse_value_arc,
           fixed_table, line_table, arc_table, W, b):
    ptr = _sc_winner(
        sparse_index_line, sparse_value_line,
        sparse_index_arc, sparse_value_arc,
    )
    b2d = b.reshape(1, D)
    fixed_proj = _project(fixed_table, W[:D], b2d, 10000)
    ext = jnp.concatenate(
        [line_table, arc_table,
         jnp.zeros((CROWS - 2 * FEAT, D), jnp.float32)], axis=0)
    combo_proj = jnp.tile(
        _project(ext, W[D:], jnp.zeros((1, D), jnp.float32), CROWS), (REP, 1))
    return _sc_emit(fixed_features, ptr, fixed_proj, combo_proj)


# speedup vs baseline: 1.0159x; 1.0159x over previous
import functools

import jax
import jax.numpy as jnp
from jax import lax
from jax.experimental import pallas as pl
from jax.experimental.pallas import tpu as pltpu
from jax.experimental.pallas import tpu_sc as plsc

N = 65536
NS_TOT = 32768
CARD = 100000
FEAT = 1000
D = 256
CROWS = 4096
REP = 4
NC = 2
NSUB = 16
NW = NC * NSUB
ROWS_PER_W = N // NW
SUB = 64
DEPTH = 3

NWP = 8
JS_PER_P = NS_TOT // NWP
BINS = N // NC // NSUB


def _proj_body(a_ref, w_ref, b_ref, o_ref):
    o_ref[...] = (
        jnp.dot(a_ref[...].astype(jnp.bfloat16),
                w_ref[...].astype(jnp.bfloat16),
                preferred_element_type=jnp.float32)
        + b_ref[...]
    )


def _project(table, w, b2d, m_blk):
    m = table.shape[0]
    return pl.pallas_call(
        _proj_body,
        grid=(m // m_blk,),
        in_specs=[
            pl.BlockSpec((m_blk, D), lambda i: (i, 0)),
            pl.BlockSpec((D, D), lambda i: (0, 0)),
            pl.BlockSpec((1, D), lambda i: (0, 0)),
        ],
        out_specs=pl.BlockSpec((m_blk, D), lambda i: (i, 0)),
        out_shape=jax.ShapeDtypeStruct((m, D), jnp.float32),
    )(table, w, b2d)


def _sc_winner_body(
    idx_line_hbm, val_line_hbm, idx_arc_hbm, val_arc_hbm,
    ptr_hbm,
    winner_v, stage_i_v, stage_v_v, parts_v, ptr_v, spmem_sh, sem_p,
):
    c = lax.axis_index("c")
    s = lax.axis_index("s")
    iota = lax.iota(jnp.int32, 16)
    neg1 = jnp.full((16,), -1, jnp.int32)

    @pl.when(s < NWP)
    def _():
        def init_body(i, _):
            for u in range(8):
                winner_v[pl.ds(i * 128 + u * 16, 16)] = neg1
            return 0

        lax.fori_loop(0, N // 128, init_body, 0)

        jbase = s * JS_PER_P

        def run_table(idx_hbm, val_hbm, table_flag):
            pltpu.sync_copy(idx_hbm.at[pl.ds(jbase, JS_PER_P)], stage_i_v)
            pltpu.sync_copy(val_hbm.at[pl.ds(jbase, JS_PER_P)], stage_v_v)
            tconst = table_flag << 25

            def chunk_body(i, _):
                idx16 = stage_i_v[pl.ds(i * 16, 16)]
                val16 = stage_v_v[pl.ds(i * 16, 16)]
                jvec = jbase + i * 16 + iota
                key = jvec * 1024 + val16 + tconst
                plsc.store_scatter(winner_v, [idx16], key)
                return 0

            lax.fori_loop(0, JS_PER_P // 16, chunk_body, 0)

        run_table(idx_line_hbm, val_line_hbm, 0)
        run_table(idx_arc_hbm, val_arc_hbm, 1)

        pltpu.sync_copy(winner_v, spmem_sh.at[s])

    plsc.subcore_barrier()

    binbase = c * (N // NC) + s * BINS
    for t in range(NWP):
        pltpu.async_copy(
            spmem_sh.at[t, pl.ds(binbase, BINS)], parts_v.at[t], sem_p)
    for t in range(NWP):
        pltpu.make_async_copy(
            spmem_sh.at[t, pl.ds(binbase, BINS)], parts_v.at[t], sem_p).wait()

    def dec_body(i, _):
        sl = pl.ds(i * 16, 16)
        k16 = parts_v[0, sl]
        for t in range(1, NWP):
            k16 = jnp.maximum(k16, parts_v[t, sl])
        tab = lax.shift_right_logical(k16, 25)
        ptr = tab * FEAT + (k16 & 1023)
        zptr = 2000 + ((i * 16) & 2047) + iota
        ptr_v[sl] = jnp.where(k16 < 0, zptr, ptr) + (i & (REP - 1)) * CROWS
        return 0

    lax.fori_loop(0, BINS // 16, dec_body, 0)

    pltpu.sync_copy(ptr_v, ptr_hbm.at[pl.ds(binbase, BINS)])


_sc_winner = functools.partial(
    pl.kernel,
    out_type=jax.ShapeDtypeStruct((N,), jnp.int32),
    mesh=plsc.VectorSubcoreMesh(core_axis_name="c", subcore_axis_name="s"),
    compiler_params=pltpu.CompilerParams(needs_layout_passes=False),
    scratch_types=[
        pltpu.VMEM((N,), jnp.int32),
        pltpu.VMEM((JS_PER_P,), jnp.int32),
        pltpu.VMEM((JS_PER_P,), jnp.int32),
        pltpu.VMEM((NWP, BINS), jnp.int32),
        pltpu.VMEM((BINS,), jnp.int32),
        pltpu.VMEM_SHARED((NWP, N), jnp.int32),
        pltpu.SemaphoreType.DMA,
    ],
)(_sc_winner_body)


def _sc_emit_body(
    ff_hbm, ptr_hbm, fproj_hbm, cproj_hbm,
    out_hbm,
    ff_v, ptr_v, buf_a, buf_b,
    sem_a, sem_b, sem_o,
):
    c = lax.axis_index("c")
    s = lax.axis_index("s")
    w = s * NC + c
    rowbase = w * ROWS_PER_W

    pltpu.sync_copy(ff_hbm.at[pl.ds(rowbase, ROWS_PER_W)], ff_v)
    pltpu.sync_copy(ptr_hbm.at[pl.ds(rowbase, ROWS_PER_W)], ptr_v)

    sems_a = [sem_a.at[i] for i in range(DEPTH)]
    sems_b = [sem_b.at[i] for i in range(DEPTH)]
    sems_o = [sem_o.at[i] for i in range(DEPTH)]
    NSB = ROWS_PER_W // SUB

    def gather_pair(sb, slot):
        pltpu.async_copy(
            fproj_hbm.at[ff_v.at[pl.ds(sb * SUB, SUB)]],
            buf_a.at[slot], sems_a[slot])
        pltpu.async_copy(
            cproj_hbm.at[ptr_v.at[pl.ds(sb * SUB, SUB)]],
            buf_b.at[slot], sems_b[slot])

    def wait_pair(sb, slot):
        pltpu.make_async_copy(
            fproj_hbm.at[ff_v.at[pl.ds(sb * SUB, SUB)]],
            buf_a.at[slot], sems_a[slot]).wait()
        pltpu.make_async_copy(
            cproj_hbm.at[ptr_v.at[pl.ds(sb * SUB, SUB)]],
            buf_b.at[slot], sems_b[slot]).wait()

    def out_desc(sb, slot):
        return pltpu.make_async_copy(
            buf_a.at[slot], out_hbm.at[pl.ds(rowbase + sb * SUB, SUB)],
            sems_o[slot])

    for i in range(DEPTH - 1):
        gather_pair(i, i)
    for sb in range(NSB):
        slot = sb % DEPTH
        wait_pair(sb, slot)
        nxt = sb + DEPTH - 1
        if nxt < NSB:
            tslot = nxt % DEPTH
            if sb >= 1:
                out_desc(sb - 1, tslot).wait()
            gather_pair(nxt, tslot)

        def add_body(r, _, slot=slot):
            for kk in range(D // 16):
                sl = pl.ds(kk * 16, 16)
                plsc.addupdate(buf_a.at[slot, r, sl], buf_b[slot, r, sl])
            return 0

        lax.fori_loop(0, SUB, add_body, 0)
        out_desc(sb, slot).start()
    for m in range(NSB - DEPTH, NSB):
        out_desc(m, m % DEPTH).wait()


_sc_emit = functools.partial(
    pl.kernel,
    out_type=jax.ShapeDtypeStruct((N, D), jnp.float32),
    mesh=plsc.VectorSubcoreMesh(core_axis_name="c", subcore_axis_name="s"),
    compiler_params=pltpu.CompilerParams(needs_layout_passes=False),
    scratch_types=[
        pltpu.VMEM((ROWS_PER_W,), jnp.int32),
        pltpu.VMEM((ROWS_PER_W,), jnp.int32),
        pltpu.VMEM((DEPTH, SUB, D), jnp.float32),
        pltpu.VMEM((DEPTH, SUB, D), jnp.float32),
        pltpu.SemaphoreType.DMA((DEPTH,)),
        pltpu.SemaphoreType.DMA((DEPTH,)),
        pltpu.SemaphoreType.DMA((DEPTH,)),
    ],
)(_sc_emit_body)


def kernel(fixed_features, sparse_index_line, sparse_value_line,
           sparse_index_arc, sparse_value_arc,
           fixed_table, line_table, arc_table, W, b):
    ptr = _sc_winner(
        sparse_index_line, sparse_value_line,
        sparse_index_arc, sparse_value_arc,
    )
    b2d = b.reshape(1, D)
    fixed_proj = _project(fixed_table, W[:D], b2d, 10000)
    ext = jnp.concatenate(
        [line_table, arc_table,
         jnp.zeros((CROWS - 2 * FEAT, D), jnp.float32)], axis=0)
    combo_proj = jnp.tile(
        _project(ext, W[D:], jnp.zeros((1, D), jnp.float32), CROWS), (REP, 1))
    return _sc_emit(fixed_features, ptr, fixed_proj, combo_proj)


# speedup vs baseline: 1.0542x; 1.0377x over previous
import functools

import jax
import jax.numpy as jnp
from jax import lax
from jax.experimental import pallas as pl
from jax.experimental.pallas import tpu as pltpu
from jax.experimental.pallas import tpu_sc as plsc

N = 65536
NS_TOT = 32768
CARD = 100000
FEAT = 1000
D = 256
CROWS = 4096
REP = 1
NC = 2
NSUB = 16
NW = NC * NSUB
ROWS_PER_W = N // NW
SUB = 64
DEPTH = 3

NWP = 8
JS_PER_P = NS_TOT // NWP
BINS = N // NC // NSUB


def _proj_body(a_ref, w_ref, b_ref, o_ref):
    o_ref[...] = (
        jnp.dot(a_ref[...].astype(jnp.bfloat16),
                w_ref[...].astype(jnp.bfloat16),
                preferred_element_type=jnp.float32)
        + b_ref[...]
    )


def _project(table, w, b2d, m_blk):
    m = table.shape[0]
    return pl.pallas_call(
        _proj_body,
        grid=(m // m_blk,),
        in_specs=[
            pl.BlockSpec((m_blk, D), lambda i: (i, 0)),
            pl.BlockSpec((D, D), lambda i: (0, 0)),
            pl.BlockSpec((1, D), lambda i: (0, 0)),
        ],
        out_specs=pl.BlockSpec((m_blk, D), lambda i: (i, 0)),
        out_shape=jax.ShapeDtypeStruct((m, D), jnp.float32),
    )(table, w, b2d)


def _sc_winner_body(
    idx_line_hbm, val_line_hbm, idx_arc_hbm, val_arc_hbm,
    ptr_hbm,
    winner_v, stage_i_v, stage_v_v, parts_v, ptr_v, spmem_sh, sem_p,
):
    c = lax.axis_index("c")
    s = lax.axis_index("s")
    iota = lax.iota(jnp.int32, 16)
    neg1 = jnp.full((16,), -1, jnp.int32)

    @pl.when(s < NWP)
    def _():
        def init_body(i, _):
            for u in range(8):
                winner_v[pl.ds(i * 128 + u * 16, 16)] = neg1
            return 0

        lax.fori_loop(0, N // 128, init_body, 0)

        jbase = s * JS_PER_P

        def run_table(idx_hbm, val_hbm, table_flag):
            pltpu.sync_copy(idx_hbm.at[pl.ds(jbase, JS_PER_P)], stage_i_v)
            pltpu.sync_copy(val_hbm.at[pl.ds(jbase, JS_PER_P)], stage_v_v)
            tconst = table_flag << 25

            def chunk_body(i, _):
                idx16 = stage_i_v[pl.ds(i * 16, 16)]
                val16 = stage_v_v[pl.ds(i * 16, 16)]
                jvec = jbase + i * 16 + iota
                key = jvec * 1024 + val16 + tconst
                for k in range(16):
                    plsc.store_scatter(winner_v, [idx16], key, mask=iota == k)
                return 0

            lax.fori_loop(0, JS_PER_P // 16, chunk_body, 0)

        run_table(idx_line_hbm, val_line_hbm, 0)
        run_table(idx_arc_hbm, val_arc_hbm, 1)

        pltpu.sync_copy(winner_v, spmem_sh.at[s])

    plsc.subcore_barrier()

    binbase = c * (N // NC) + s * BINS
    for t in range(NWP):
        pltpu.async_copy(
            spmem_sh.at[t, pl.ds(binbase, BINS)], parts_v.at[t], sem_p)
    for t in range(NWP):
        pltpu.make_async_copy(
            spmem_sh.at[t, pl.ds(binbase, BINS)], parts_v.at[t], sem_p).wait()

    def dec_body(i, _):
        sl = pl.ds(i * 16, 16)
        k16 = parts_v[0, sl]
        for t in range(1, NWP):
            k16 = jnp.maximum(k16, parts_v[t, sl])
        tab = lax.shift_right_logical(k16, 25)
        ptr = tab * FEAT + (k16 & 1023)
        zptr = 2000 + ((i * 16) & 2047) + iota
        ptr_v[sl] = jnp.where(k16 < 0, zptr, ptr) + (i & (REP - 1)) * CROWS
        return 0

    lax.fori_loop(0, BINS // 16, dec_body, 0)

    pltpu.sync_copy(ptr_v, ptr_hbm.at[pl.ds(binbase, BINS)])


_sc_winner = functools.partial(
    pl.kernel,
    out_type=jax.ShapeDtypeStruct((N,), jnp.int32),
    mesh=plsc.VectorSubcoreMesh(core_axis_name="c", subcore_axis_name="s"),
    compiler_params=pltpu.CompilerParams(needs_layout_passes=False),
    scratch_types=[
        pltpu.VMEM((N,), jnp.int32),
        pltpu.VMEM((JS_PER_P,), jnp.int32),
        pltpu.VMEM((JS_PER_P,), jnp.int32),
        pltpu.VMEM((NWP, BINS), jnp.int32),
        pltpu.VMEM((BINS,), jnp.int32),
        pltpu.VMEM_SHARED((NWP, N), jnp.int32),
        pltpu.SemaphoreType.DMA,
    ],
)(_sc_winner_body)


def _sc_emit_body(
    ff_hbm, ptr_hbm, fproj_hbm, cproj_hbm,
    out_hbm,
    ff_v, ptr_v, buf_a, buf_b,
    sem_a, sem_b, sem_o,
):
    c = lax.axis_index("c")
    s = lax.axis_index("s")
    w = s * NC + c
    rowbase = w * ROWS_PER_W

    pltpu.sync_copy(ff_hbm.at[pl.ds(rowbase, ROWS_PER_W)], ff_v)
    pltpu.sync_copy(ptr_hbm.at[pl.ds(rowbase, ROWS_PER_W)], ptr_v)

    sems_a = [sem_a.at[i] for i in range(DEPTH)]
    sems_b = [sem_b.at[i] for i in range(DEPTH)]
    sems_o = [sem_o.at[i] for i in range(DEPTH)]
    NSB = ROWS_PER_W // SUB

    def gather_pair(sb, slot):
        pltpu.async_copy(
            fproj_hbm.at[ff_v.at[pl.ds(sb * SUB, SUB)]],
            buf_a.at[slot], sems_a[slot])
        pltpu.async_copy(
            cproj_hbm.at[ptr_v.at[pl.ds(sb * SUB, SUB)]],
            buf_b.at[slot], sems_b[slot])

    def wait_pair(sb, slot):
        pltpu.make_async_copy(
            fproj_hbm.at[ff_v.at[pl.ds(sb * SUB, SUB)]],
            buf_a.at[slot], sems_a[slot]).wait()
        pltpu.make_async_copy(
            cproj_hbm.at[ptr_v.at[pl.ds(sb * SUB, SUB)]],
            buf_b.at[slot], sems_b[slot]).wait()

    def out_desc(sb, slot):
        return pltpu.make_async_copy(
            buf_a.at[slot], out_hbm.at[pl.ds(rowbase + sb * SUB, SUB)],
            sems_o[slot])

    for i in range(DEPTH - 1):
        gather_pair(i, i)
    for sb in range(NSB):
        slot = sb % DEPTH
        wait_pair(sb, slot)
        nxt = sb + DEPTH - 1
        if nxt < NSB:
            tslot = nxt % DEPTH
            if sb >= 1:
                out_desc(sb - 1, tslot).wait()
            gather_pair(nxt, tslot)

        def add_body(r, _, slot=slot):
            for kk in range(D // 16):
                sl = pl.ds(kk * 16, 16)
                plsc.addupdate(buf_a.at[slot, r, sl], buf_b[slot, r, sl])
            return 0

        lax.fori_loop(0, SUB, add_body, 0)
        out_desc(sb, slot).start()
    for m in range(NSB - DEPTH, NSB):
        out_desc(m, m % DEPTH).wait()


_sc_emit = functools.partial(
    pl.kernel,
    out_type=jax.ShapeDtypeStruct((N, D), jnp.float32),
    mesh=plsc.VectorSubcoreMesh(core_axis_name="c", subcore_axis_name="s"),
    compiler_params=pltpu.CompilerParams(needs_layout_passes=False),
    scratch_types=[
        pltpu.VMEM((ROWS_PER_W,), jnp.int32),
        pltpu.VMEM((ROWS_PER_W,), jnp.int32),
        pltpu.VMEM((DEPTH, SUB, D), jnp.float32),
        pltpu.VMEM((DEPTH, SUB, D), jnp.float32),
        pltpu.SemaphoreType.DMA((DEPTH,)),
        pltpu.SemaphoreType.DMA((DEPTH,)),
        pltpu.SemaphoreType.DMA((DEPTH,)),
    ],
)(_sc_emit_body)


def kernel(fixed_features, sparse_index_line, sparse_value_line,
           sparse_index_arc, sparse_value_arc,
           fixed_table, line_table, arc_table, W, b):
    ptr = _sc_winner(
        sparse_index_line, sparse_value_line,
        sparse_index_arc, sparse_value_arc,
    )
    b2d = b.reshape(1, D)
    fixed_proj = _project(fixed_table, W[:D], b2d, 10000)
    ext = jnp.concatenate(
        [line_table, arc_table,
         jnp.zeros((CROWS - 2 * FEAT, D), jnp.float32)], axis=0)
    combo_proj = jnp.tile(
        _project(ext, W[D:], jnp.zeros((1, D), jnp.float32), CROWS), (REP, 1))
    return _sc_emit(fixed_features, ptr, fixed_proj, combo_proj)
